# baseline (speedup 1.0000x reference)
import functools

import jax
import jax.numpy as jnp
from jax import lax
from jax.experimental import pallas as pl
from jax.experimental.pallas import tpu as pltpu
from jax.experimental.pallas import tpu_sc as plsc

_DIM = 4096
_BATCH = 8192
_CTRL_MASK = 2048 | 64

_NW = 32
_RPW = _BATCH // _NW
_CR = 8
_NCHUNK = _RPW // _CR
_CHUNK = _CR * _DIM

_SEG_STARTS = [2048 + 64 + 128 * k for k in range(16)]

_mesh = plsc.VectorSubcoreMesh(core_axis_name="c", subcore_axis_name="s")


@functools.partial(
    pl.kernel,
    mesh=_mesh,
    out_type=jax.ShapeDtypeStruct((_BATCH * _DIM,), jnp.float32),
    scratch_types=[pltpu.VMEM((_CHUNK + 8,), jnp.float32)],
)
def _sc_perm(state_hbm, out_hbm, buf):
    wid = lax.axis_index("s") * 2 + lax.axis_index("c")
    base = wid * _RPW * _DIM
    odd = (lax.iota(jnp.int32, 16) & 1) != 0

    def chunk_body(ci, _):
        off = base + ci * _CHUNK
        pltpu.sync_copy(state_hbm.at[pl.ds(off, _CHUNK)], buf.at[pl.ds(0, _CHUNK)])

        def row_body(r, _):
            rbase = r * _DIM
            for seg in _SEG_STARTS:
                for v in range(4):
                    c = rbase + seg + 16 * v
                    a = buf[pl.ds(c - 1, 16)]
                    b = buf[pl.ds(c + 1, 16)]
                    buf[pl.ds(c, 16)] = jnp.where(odd, a, b)
            return 0

        lax.fori_loop(0, _CR, row_body, 0)
        pltpu.sync_copy(buf.at[pl.ds(0, _CHUNK)], out_hbm.at[pl.ds(off, _CHUNK)])
        return 0

    lax.fori_loop(0, _NCHUNK, chunk_body, 0)


def kernel(state, M):
    del M
    out = _sc_perm(state.reshape(_BATCH * _DIM))
    return out.reshape(_BATCH, _DIM)


# speedup vs baseline: 1.1108x; 1.1108x over previous
import functools

import jax
import jax.numpy as jnp
from jax import lax
from jax.experimental import pallas as pl
from jax.experimental.pallas import tpu as pltpu
from jax.experimental.pallas import tpu_sc as plsc

_DIM = 4096
_BATCH = 8192
_CTRL_MASK = 2048 | 64

_NW = 32
_RPW = _BATCH // _NW
_CR = 4
_NCHUNK = _RPW // _CR
_CHUNK = _CR * _DIM
_NB = 4
_NGROUP = _NCHUNK // _NB

_SEG_STARTS = [2048 + 64 + 128 * k for k in range(16)]

_mesh = plsc.VectorSubcoreMesh(core_axis_name="c", subcore_axis_name="s")

_SCRATCH = [pltpu.VMEM((_CHUNK + 8,), jnp.float32) for _ in range(_NB)]
_SCRATCH += [pltpu.SemaphoreType.DMA for _ in range(2 * _NB)]


@functools.partial(
    pl.kernel,
    mesh=_mesh,
    out_type=jax.ShapeDtypeStruct((_BATCH * _DIM,), jnp.float32),
    scratch_types=_SCRATCH,
)
def _sc_perm(state_hbm, out_hbm, *scratch):
    bufs = scratch[:_NB]
    isems = scratch[_NB:2 * _NB]
    osems = scratch[2 * _NB:]
    wid = lax.axis_index("s") * 2 + lax.axis_index("c")
    base = wid * _RPW * _DIM
    odd = (lax.iota(jnp.int32, 16) & 1) != 0

    def in_copy(ci, b):
        return pltpu.make_async_copy(
            state_hbm.at[pl.ds(base + ci * _CHUNK, _CHUNK)],
            bufs[b].at[pl.ds(0, _CHUNK)],
            isems[b],
        )

    def out_copy(ci, b):
        return pltpu.make_async_copy(
            bufs[b].at[pl.ds(0, _CHUNK)],
            out_hbm.at[pl.ds(base + ci * _CHUNK, _CHUNK)],
            osems[b],
        )

    for b in range(_NB):
        in_copy(b, b).start()

    def group_body(g, _):
        for b in range(_NB):
            ci = g * _NB + b
            in_copy(ci, b).wait()

            def row_body(r, _):
                rbase = r * _DIM
                for seg in _SEG_STARTS:
                    for v in range(4):
                        c = rbase + seg + 16 * v
                        a = bufs[b][pl.ds(c - 1, 16)]
                        bb = bufs[b][pl.ds(c + 1, 16)]
                        bufs[b][pl.ds(c, 16)] = jnp.where(odd, a, bb)
                return 0

            lax.fori_loop(0, _CR, row_body, 0)
            out_copy(ci, b).start()
            out_copy(ci, b).wait()

            @pl.when(g < _NGROUP - 1)
            def _():
                in_copy(ci + _NB, b).start()

        return 0

    lax.fori_loop(0, _NGROUP, group_body, 0)


def kernel(state, M):
    del M
    out = _sc_perm(state.reshape(_BATCH * _DIM))
    return out.reshape(_BATCH, _DIM)
